# baseline (speedup 1.0000x reference)
import functools

import jax
import jax.numpy as jnp
from jax import lax
from jax.experimental import pallas as pl
from jax.experimental.pallas import tpu as pltpu
from jax.experimental.pallas import tpu_sc as plsc

N_NODES = 10000
N_EDGES = 320000
D = 128
N_GRAPHS = 128
N_CLASSES = 6
N_LAYERS = 5

NSC = 2
NTILES = 16
NWKR = NSC * NTILES
EDGES_PER_TILE = N_EDGES // NWKR
EDGE_WIN = 80
EDGES_PER_TILE_PAD = 10240
N_WIN = EDGES_PER_TILE_PAD // EDGE_WIN
N_PAD = 10240
ROWS_PER_TILE = N_PAD // NTILES
ZERO_ROWS = 80

ROW_BLK = 1000
N_BLKS = N_NODES // ROW_BLK


def _sc_edge_scatter(h, src2, dst2):
    mesh = plsc.VectorSubcoreMesh(core_axis_name="c", subcore_axis_name="s")

    scratch = ([pltpu.VMEM((EDGE_WIN,), jnp.int32) for _ in range(8)]
               + [pltpu.VMEM((EDGE_WIN, D), jnp.float32) for _ in range(2)]
               + [pltpu.VMEM_SHARED((N_PAD, D), jnp.float32)]
               + [pltpu.SemaphoreType.DMA for _ in range(4)])

    @functools.partial(
        pl.kernel,
        out_type=jax.ShapeDtypeStruct((NSC * N_PAD, D), jnp.float32),
        mesh=mesh,
        scratch_types=scratch,
    )
    def scatter_kernel(h_hbm, src_hbm, dst_hbm, out_hbm, *refs):
        sidx = refs[0:4]
        didx = refs[4:8]
        rows = refs[8:10]
        agg_sh = refs[10]
        gsem = refs[11:13]
        ssem = refs[13:15]

        cid = lax.axis_index("c")
        sid = lax.axis_index("s")
        wid = cid * NTILES + sid
        ebase = wid * EDGES_PER_TILE_PAD

        zv = jnp.zeros((16,), jnp.float32)

        @pl.loop(0, EDGE_WIN)
        def _(r):
            for c in range(0, D, 16):
                rows[0][r, pl.ds(c, 16)] = zv

        @pl.loop(0, ROWS_PER_TILE // EDGE_WIN)
        def _(j):
            pltpu.sync_copy(
                rows[0],
                agg_sh.at[pl.ds(sid * ROWS_PER_TILE + j * EDGE_WIN, EDGE_WIN)])

        def load_idx(j, w):
            e0 = ebase + w * EDGE_WIN
            pltpu.sync_copy(src_hbm.at[pl.ds(e0, EDGE_WIN)], sidx[j])
            pltpu.sync_copy(dst_hbm.at[pl.ds(e0, EDGE_WIN)], didx[j])

        def gather(j, b):
            pltpu.async_copy(h_hbm.at[sidx[j]], rows[b], gsem[b])

        def wait_gather(j, b):
            pltpu.make_async_copy(h_hbm.at[sidx[j]], rows[b], gsem[b]).wait()

        def scatter(j, b):
            pltpu.async_copy(rows[b], agg_sh.at[didx[j]], ssem[b], add=True)

        def wait_scatter(j, b):
            pltpu.make_async_copy(rows[b], agg_sh.at[didx[j]], ssem[b]).wait()

        for j in range(4):
            load_idx(j, j)
        gather(0, 0)
        gather(1, 1)

        plsc.subcore_barrier()

        @pl.loop(0, N_WIN, step=4)
        def _(w):
            wait_gather(0, 0)
            scatter(0, 0)
            wait_gather(1, 1)
            scatter(1, 1)
            wait_scatter(0, 0)
            gather(2, 0)

            @pl.when(w + 4 < N_WIN)
            def _():
                load_idx(0, w + 4)

            wait_scatter(1, 1)
            gather(3, 1)

            @pl.when(w + 5 < N_WIN)
            def _():
                load_idx(1, w + 5)

            wait_gather(2, 0)
            scatter(2, 0)
            wait_gather(3, 1)
            scatter(3, 1)
            wait_scatter(2, 0)

            @pl.when(w + 4 < N_WIN)
            def _():
                gather(0, 0)

            @pl.when(w + 6 < N_WIN)
            def _():
                load_idx(2, w + 6)

            wait_scatter(3, 1)

            @pl.when(w + 5 < N_WIN)
            def _():
                gather(1, 1)

            @pl.when(w + 7 < N_WIN)
            def _():
                load_idx(3, w + 7)

        plsc.subcore_barrier()
        r0 = sid * ROWS_PER_TILE
        pltpu.sync_copy(
            agg_sh.at[pl.ds(r0, ROWS_PER_TILE)],
            out_hbm.at[pl.ds(cid * N_PAD + r0, ROWS_PER_TILE)])

    return scatter_kernel(h, src2, dst2)


def _encoder_body(x_ref, w_ref, b_ref, o_ref):
    z = jnp.dot(x_ref[...], w_ref[...], preferred_element_type=jnp.float32)
    o_ref[...] = jnp.maximum(z + b_ref[...], 0.0)


def _tc_encoder(x, W, b):
    return pl.pallas_call(
        _encoder_body,
        grid=(N_BLKS,),
        in_specs=[
            pl.BlockSpec((ROW_BLK, D), lambda i: (i, 0)),
            pl.BlockSpec((D, D), lambda i: (0, 0)),
            pl.BlockSpec((1, D), lambda i: (0, 0)),
        ],
        out_specs=pl.BlockSpec((ROW_BLK, D), lambda i: (i, 0)),
        out_shape=jax.ShapeDtypeStruct((N_NODES, D), jnp.float32),
    )(x, W, b)


def _gin_body(scale_ref, h_ref, agg_ref, w1_ref, b1_ref, w2_ref, b2_ref, o_ref):
    z = scale_ref[...] * h_ref[...] + agg_ref[0] + agg_ref[1]
    z = jnp.maximum(
        jnp.dot(z, w1_ref[...], preferred_element_type=jnp.float32) + b1_ref[...],
        0.0)
    z = jnp.dot(z, w2_ref[...], preferred_element_type=jnp.float32) + b2_ref[...]
    o_ref[...] = jnp.maximum(z, 0.0)


def _tc_gin_layer(h, agg2, scale, W1, b1, W2, b2):
    return pl.pallas_call(
        _gin_body,
        grid=(N_BLKS,),
        in_specs=[
            pl.BlockSpec((1, D), lambda i: (0, 0)),
            pl.BlockSpec((ROW_BLK, D), lambda i: (i, 0)),
            pl.BlockSpec((NSC, ROW_BLK, D), lambda i: (0, i, 0)),
            pl.BlockSpec((D, D), lambda i: (0, 0)),
            pl.BlockSpec((1, D), lambda i: (0, 0)),
            pl.BlockSpec((D, D), lambda i: (0, 0)),
            pl.BlockSpec((1, D), lambda i: (0, 0)),
        ],
        out_specs=pl.BlockSpec((ROW_BLK, D), lambda i: (i, 0)),
        out_shape=jax.ShapeDtypeStruct((N_NODES, D), jnp.float32),
    )(scale, h, agg2, W1, b1, W2, b2)


def _pool_head_body(h_ref, b_ref, wh1_ref, bh1_ref, wh2_ref, bh2_ref,
                    o_ref, sums, counts):
    i = pl.program_id(0)

    @pl.when(i == 0)
    def _():
        sums[...] = jnp.zeros_like(sums)
        counts[...] = jnp.zeros_like(counts)

    gid = lax.broadcasted_iota(jnp.int32, (ROW_BLK, N_GRAPHS), 1)
    onehot = (b_ref[...] == gid).astype(jnp.float32)
    dn = (((0,), (0,)), ((), ()))
    sums[...] += lax.dot_general(onehot, h_ref[...], dn,
                                 preferred_element_type=jnp.float32)
    counts[...] += lax.dot_general(onehot, jnp.ones((ROW_BLK, D), jnp.float32),
                                   dn, preferred_element_type=jnp.float32)

    @pl.when(i == N_BLKS - 1)
    def _():
        g = sums[...] / jnp.maximum(counts[...], 1.0)
        t = jnp.maximum(
            jnp.dot(g, wh1_ref[...], preferred_element_type=jnp.float32)
            + bh1_ref[...], 0.0)
        o_ref[...] = (jnp.dot(t, wh2_ref[...], preferred_element_type=jnp.float32)
                      + bh2_ref[...])


def _tc_pool_head(h, batch_b, Wh1, bh1, Wh2p, bh2p):
    return pl.pallas_call(
        _pool_head_body,
        grid=(N_BLKS,),
        in_specs=[
            pl.BlockSpec((ROW_BLK, D), lambda i: (i, 0)),
            pl.BlockSpec((ROW_BLK, N_GRAPHS), lambda i: (i, 0)),
            pl.BlockSpec((D, D), lambda i: (0, 0)),
            pl.BlockSpec((1, D), lambda i: (0, 0)),
            pl.BlockSpec((D, D), lambda i: (0, 0)),
            pl.BlockSpec((1, D), lambda i: (0, 0)),
        ],
        out_specs=pl.BlockSpec((N_GRAPHS, D), lambda i: (0, 0)),
        out_shape=jax.ShapeDtypeStruct((N_GRAPHS, D), jnp.float32),
        scratch_shapes=[
            pltpu.VMEM((N_GRAPHS, D), jnp.float32),
            pltpu.VMEM((N_GRAPHS, D), jnp.float32),
        ],
    )(h, batch_b, Wh1, bh1, Wh2p, bh2p)


def kernel(x, edge_index, batch, W_in, b_in, W1s, b1s, W2s, b2s, eps,
           Wh1, bh1, Wh2, bh2):
    pad = EDGES_PER_TILE_PAD - EDGES_PER_TILE
    src2 = jnp.pad(edge_index[0].reshape(NWKR, EDGES_PER_TILE), ((0, 0), (0, pad)),
                   constant_values=0).reshape(NWKR * EDGES_PER_TILE_PAD)
    pad_dst = jnp.broadcast_to(N_NODES + jnp.arange(pad, dtype=jnp.int32),
                               (NWKR, pad))
    dst2 = jnp.concatenate(
        [edge_index[1].reshape(NWKR, EDGES_PER_TILE), pad_dst],
        axis=1).reshape(NWKR * EDGES_PER_TILE_PAD)
    h = _tc_encoder(x, W_in, b_in.reshape(1, D))
    for l in range(N_LAYERS):
        aggf = _sc_edge_scatter(h, src2, dst2).reshape(NSC, N_PAD, D)
        scale = (1.0 + eps[l]) * jnp.ones((1, D), jnp.float32)
        h = _tc_gin_layer(h, aggf, scale, W1s[l], b1s[l].reshape(1, D),
                          W2s[l], b2s[l].reshape(1, D))
    batch_b = jnp.broadcast_to(batch[:, None], (N_NODES, N_GRAPHS))
    Wh2p = jnp.zeros((D, D), jnp.float32).at[:, :N_CLASSES].set(Wh2)
    bh2p = jnp.zeros((1, D), jnp.float32).at[0, :N_CLASSES].set(bh2)
    out = _tc_pool_head(h, batch_b, Wh1, bh1.reshape(1, D), Wh2p, bh2p)
    return out[:, :N_CLASSES]


# speedup vs baseline: 2.4748x; 2.4748x over previous
import functools

import jax
import jax.numpy as jnp
from jax import lax
from jax.experimental import pallas as pl
from jax.experimental.pallas import tpu as pltpu
from jax.experimental.pallas import tpu_sc as plsc

N_NODES = 10000
N_EDGES = 320000
D = 128
N_GRAPHS = 128
N_CLASSES = 6
N_LAYERS = 5

NSC = 2
NTILES = 16
NWKR = NSC * NTILES
EDGES_PER_TILE = N_EDGES // NWKR
EDGE_WIN = 80
EDGES_PER_TILE_PAD = 10000
N_WIN = EDGES_PER_TILE_PAD // EDGE_WIN
N_PAD = 10240
ROWS_PER_TILE = N_PAD // NTILES
ZERO_ROWS = 80

ROW_BLK = 1000
N_BLKS = N_NODES // ROW_BLK


def _sc_edge_scatter(h, src2, dst2):
    mesh = plsc.VectorSubcoreMesh(core_axis_name="c", subcore_axis_name="s")

    scratch = ([pltpu.VMEM((EDGE_WIN,), jnp.int32) for _ in range(8)]
               + [pltpu.VMEM((EDGE_WIN, D), jnp.float32) for _ in range(2)]
               + [pltpu.VMEM_SHARED((N_PAD, D), jnp.float32)]
               + [pltpu.SemaphoreType.DMA for _ in range(4)])

    @functools.partial(
        pl.kernel,
        out_type=jax.ShapeDtypeStruct((NSC * N_PAD, D), jnp.float32),
        mesh=mesh,
        scratch_types=scratch,
    )
    def scatter_kernel(h_hbm, src_hbm, dst_hbm, out_hbm, *refs):
        sidx = refs[0:4]
        didx = refs[4:8]
        rows = refs[8:10]
        agg_sh = refs[10]
        gsem = refs[11:13]
        ssem = refs[13:15]

        cid = lax.axis_index("c")
        sid = lax.axis_index("s")
        wid = cid * NTILES + sid
        ebase = wid * EDGES_PER_TILE_PAD

        zv = jnp.zeros((16,), jnp.float32)

        @pl.loop(0, EDGE_WIN)
        def _(r):
            for c in range(0, D, 16):
                rows[0][r, pl.ds(c, 16)] = zv

        @pl.loop(0, ROWS_PER_TILE // EDGE_WIN)
        def _(j):
            pltpu.sync_copy(
                rows[0],
                agg_sh.at[pl.ds(sid * ROWS_PER_TILE + j * EDGE_WIN, EDGE_WIN)])

        def load_idx(j, w):
            e0 = ebase + w * EDGE_WIN
            pltpu.sync_copy(src_hbm.at[pl.ds(e0, EDGE_WIN)], sidx[j])
            pltpu.sync_copy(dst_hbm.at[pl.ds(e0, EDGE_WIN)], didx[j])

        def gather(j, b):
            pltpu.async_copy(h_hbm.at[sidx[j]], rows[b], gsem[b])

        def wait_gather(j, b):
            pltpu.make_async_copy(h_hbm.at[sidx[j]], rows[b], gsem[b]).wait()

        def scatter(j, b):
            pltpu.async_copy(rows[b], agg_sh.at[didx[j]], ssem[b], add=True)

        def wait_scatter(j, b):
            pltpu.make_async_copy(rows[b], agg_sh.at[didx[j]], ssem[b]).wait()

        for j in range(4):
            load_idx(j, j)
        gather(0, 0)
        gather(1, 1)

        plsc.subcore_barrier()

        MAIN = (N_WIN // 4) * 4

        @pl.loop(0, MAIN, step=4)
        def _(w):
            wait_gather(0, 0)
            scatter(0, 0)
            wait_gather(1, 1)
            scatter(1, 1)
            wait_scatter(0, 0)
            gather(2, 0)

            @pl.when(w + 4 < N_WIN)
            def _():
                load_idx(0, w + 4)

            wait_scatter(1, 1)
            gather(3, 1)

            @pl.when(w + 5 < N_WIN)
            def _():
                load_idx(1, w + 5)

            wait_gather(2, 0)
            scatter(2, 0)
            wait_gather(3, 1)
            scatter(3, 1)
            wait_scatter(2, 0)

            @pl.when(w + 4 < N_WIN)
            def _():
                gather(0, 0)

            @pl.when(w + 6 < N_WIN)
            def _():
                load_idx(2, w + 6)

            wait_scatter(3, 1)

            @pl.when(w + 5 < N_WIN)
            def _():
                gather(1, 1)

            @pl.when(w + 7 < N_WIN)
            def _():
                load_idx(3, w + 7)

        for k in range(N_WIN - (N_WIN // 4) * 4):
            wait_gather(k, k % 2)
            pltpu.sync_copy(rows[k % 2], agg_sh.at[didx[k]], add=True)

        plsc.subcore_barrier()
        r0 = sid * ROWS_PER_TILE
        pltpu.sync_copy(
            agg_sh.at[pl.ds(r0, ROWS_PER_TILE)],
            out_hbm.at[pl.ds(cid * N_PAD + r0, ROWS_PER_TILE)])

    return scatter_kernel(h, src2, dst2)


def _encoder_body(x_ref, w_ref, b_ref, o_ref):
    z = jnp.dot(x_ref[...], w_ref[...], preferred_element_type=jnp.float32)
    o_ref[...] = jnp.maximum(z + b_ref[...], 0.0)


def _tc_encoder(x, W, b):
    return pl.pallas_call(
        _encoder_body,
        grid=(N_BLKS,),
        in_specs=[
            pl.BlockSpec((ROW_BLK, D), lambda i: (i, 0)),
            pl.BlockSpec((D, D), lambda i: (0, 0)),
            pl.BlockSpec((1, D), lambda i: (0, 0)),
        ],
        out_specs=pl.BlockSpec((ROW_BLK, D), lambda i: (i, 0)),
        out_shape=jax.ShapeDtypeStruct((N_NODES, D), jnp.float32),
    )(x, W, b)


def _gin_body(scale_ref, h_ref, agg_ref, w1_ref, b1_ref, w2_ref, b2_ref, o_ref):
    z = scale_ref[...] * h_ref[...] + agg_ref[0] + agg_ref[1]
    z = jnp.maximum(
        jnp.dot(z, w1_ref[...], preferred_element_type=jnp.float32) + b1_ref[...],
        0.0)
    z = jnp.dot(z, w2_ref[...], preferred_element_type=jnp.float32) + b2_ref[...]
    o_ref[...] = jnp.maximum(z, 0.0)


def _tc_gin_layer(h, agg2, scale, W1, b1, W2, b2):
    return pl.pallas_call(
        _gin_body,
        grid=(N_BLKS,),
        in_specs=[
            pl.BlockSpec((1, D), lambda i: (0, 0)),
            pl.BlockSpec((ROW_BLK, D), lambda i: (i, 0)),
            pl.BlockSpec((NSC, ROW_BLK, D), lambda i: (0, i, 0)),
            pl.BlockSpec((D, D), lambda i: (0, 0)),
            pl.BlockSpec((1, D), lambda i: (0, 0)),
            pl.BlockSpec((D, D), lambda i: (0, 0)),
            pl.BlockSpec((1, D), lambda i: (0, 0)),
        ],
        out_specs=pl.BlockSpec((ROW_BLK, D), lambda i: (i, 0)),
        out_shape=jax.ShapeDtypeStruct((N_NODES, D), jnp.float32),
    )(scale, h, agg2, W1, b1, W2, b2)


def _pool_head_body(h_ref, b_ref, wh1_ref, bh1_ref, wh2_ref, bh2_ref,
                    o_ref, sums, counts):
    i = pl.program_id(0)

    @pl.when(i == 0)
    def _():
        sums[...] = jnp.zeros_like(sums)
        counts[...] = jnp.zeros_like(counts)

    gid = lax.broadcasted_iota(jnp.int32, (ROW_BLK, N_GRAPHS), 1)
    onehot = (b_ref[...] == gid).astype(jnp.float32)
    dn = (((0,), (0,)), ((), ()))
    sums[...] += lax.dot_general(onehot, h_ref[...], dn,
                                 preferred_element_type=jnp.float32)
    counts[...] += lax.dot_general(onehot, jnp.ones((ROW_BLK, D), jnp.float32),
                                   dn, preferred_element_type=jnp.float32)

    @pl.when(i == N_BLKS - 1)
    def _():
        g = sums[...] / jnp.maximum(counts[...], 1.0)
        t = jnp.maximum(
            jnp.dot(g, wh1_ref[...], preferred_element_type=jnp.float32)
            + bh1_ref[...], 0.0)
        o_ref[...] = (jnp.dot(t, wh2_ref[...], preferred_element_type=jnp.float32)
                      + bh2_ref[...])


def _tc_pool_head(h, batch_b, Wh1, bh1, Wh2p, bh2p):
    return pl.pallas_call(
        _pool_head_body,
        grid=(N_BLKS,),
        in_specs=[
            pl.BlockSpec((ROW_BLK, D), lambda i: (i, 0)),
            pl.BlockSpec((ROW_BLK, N_GRAPHS), lambda i: (i, 0)),
            pl.BlockSpec((D, D), lambda i: (0, 0)),
            pl.BlockSpec((1, D), lambda i: (0, 0)),
            pl.BlockSpec((D, D), lambda i: (0, 0)),
            pl.BlockSpec((1, D), lambda i: (0, 0)),
        ],
        out_specs=pl.BlockSpec((N_GRAPHS, D), lambda i: (0, 0)),
        out_shape=jax.ShapeDtypeStruct((N_GRAPHS, D), jnp.float32),
        scratch_shapes=[
            pltpu.VMEM((N_GRAPHS, D), jnp.float32),
            pltpu.VMEM((N_GRAPHS, D), jnp.float32),
        ],
    )(h, batch_b, Wh1, bh1, Wh2p, bh2p)


def kernel(x, edge_index, batch, W_in, b_in, W1s, b1s, W2s, b2s, eps,
           Wh1, bh1, Wh2, bh2):
    pad = EDGES_PER_TILE_PAD - EDGES_PER_TILE
    src2 = jnp.pad(edge_index[0].reshape(NWKR, EDGES_PER_TILE), ((0, 0), (0, pad)),
                   constant_values=0).reshape(NWKR * EDGES_PER_TILE_PAD)
    pad_dst = jnp.broadcast_to(N_NODES + jnp.arange(pad, dtype=jnp.int32),
                               (NWKR, pad))
    dst2 = jnp.concatenate(
        [edge_index[1].reshape(NWKR, EDGES_PER_TILE), pad_dst],
        axis=1).reshape(NWKR * EDGES_PER_TILE_PAD)
    h = _tc_encoder(x, W_in, b_in.reshape(1, D))
    for l in range(N_LAYERS):
        aggf = _sc_edge_scatter(h, src2, dst2).reshape(NSC, N_PAD, D)
        scale = (1.0 + eps[l]) * jnp.ones((1, D), jnp.float32)
        h = _tc_gin_layer(h, aggf, scale, W1s[l], b1s[l].reshape(1, D),
                          W2s[l], b2s[l].reshape(1, D))
    batch_b = jnp.broadcast_to(batch[:, None], (N_NODES, N_GRAPHS))
    Wh2p = jnp.zeros((D, D), jnp.float32).at[:, :N_CLASSES].set(Wh2)
    bh2p = jnp.zeros((1, D), jnp.float32).at[0, :N_CLASSES].set(bh2)
    out = _tc_pool_head(h, batch_b, Wh1, bh1.reshape(1, D), Wh2p, bh2p)
    return out[:, :N_CLASSES]
